# baseline (speedup 1.0000x reference)
import functools

import jax
import jax.numpy as jnp
from jax import lax
from jax.experimental import pallas as pl
from jax.experimental.pallas import tpu as pltpu


def _round_up(x, m):
    return (x + m - 1) // m * m


_TILE = 512
_UNROLL = 8


def _bigram_kernel(idx_sref, tgt_ref, emb_hbm, logits_hbm, rowloss_ref,
                   tab, buf, tab_sem, out_sem):
    i = pl.program_id(0)
    j = pl.program_id(1)
    nj = pl.num_programs(1)
    tile = buf.shape[0]
    t = i * nj + j
    base = t * tile

    @pl.when(j == 0)
    def _load_table():
        cp = pltpu.make_async_copy(emb_hbm, tab.at[:, 0, :], tab_sem)
        cp.start()
        cp.wait()

    def _chunk(o, carry):
        for u in range(_UNROLL):
            m = o * _UNROLL + u
            buf[m] = tab[idx_sref[base + m]]
        return carry

    lax.fori_loop(0, tile // _UNROLL, _chunk, 0)

    out_cp = pltpu.make_async_copy(
        buf.at[:, 0, :], logits_hbm.at[pl.ds(base, tile), :], out_sem)
    out_cp.start()

    x = buf[...]
    mx = jnp.max(x, axis=2, keepdims=True)
    lse = mx + jnp.log(jnp.sum(jnp.exp(x - mx), axis=2, keepdims=True))
    col = lax.broadcasted_iota(jnp.int32, x.shape, 2)
    tgt = tgt_ref[...][:, :, None]
    tgt_logit = jnp.sum(jnp.where(col == tgt, x, 0.0), axis=2, keepdims=True)
    rowloss_ref[...] = (lse - tgt_logit)[:, 0, :]

    out_cp.wait()


@jax.jit
def _bigram_train(emb, idx_flat, tgt_flat):
    bt = idx_flat.shape[0]
    v = emb.shape[1]
    bt_pad = _round_up(bt, 2 * _TILE)
    nt = bt_pad // _TILE
    nj = nt // 2

    idx_pad = jnp.zeros((bt_pad,), jnp.int32).at[:bt].set(idx_flat)
    tgt_pad = jnp.zeros((bt_pad, 1), jnp.int32).at[:bt, 0].set(tgt_flat)

    grid_spec = pltpu.PrefetchScalarGridSpec(
        num_scalar_prefetch=1,
        grid=(2, nj),
        in_specs=[
            pl.BlockSpec((_TILE, 1), lambda i, j, idx: (i * nj + j, 0)),
            pl.BlockSpec(memory_space=pl.ANY),
        ],
        out_specs=[
            pl.BlockSpec(memory_space=pl.ANY),
            pl.BlockSpec((_TILE, 1), lambda i, j, idx: (i * nj + j, 0)),
        ],
        scratch_shapes=[
            pltpu.VMEM((v, 1, v), jnp.float32),
            pltpu.VMEM((_TILE, 1, v), jnp.float32),
            pltpu.SemaphoreType.DMA,
            pltpu.SemaphoreType.DMA,
        ],
    )

    logits, rowloss = pl.pallas_call(
        _bigram_kernel,
        grid_spec=grid_spec,
        out_shape=(
            jax.ShapeDtypeStruct((bt_pad, v), emb.dtype),
            jax.ShapeDtypeStruct((bt_pad, 1), jnp.float32),
        ),
        compiler_params=pltpu.CompilerParams(
            dimension_semantics=("parallel", "arbitrary"),
            vmem_limit_bytes=44 * 1024 * 1024,
        ),
    )(idx_pad, tgt_pad, emb)

    loss = jnp.sum(rowloss[:bt, 0]) / bt
    return logits[:bt], loss


def kernel(emb, idx, targets):
    b, tseq = idx.shape
    v = emb.shape[1]
    idx_flat = idx.reshape(b * tseq).astype(jnp.int32)
    if targets is None:
        logits, _ = _bigram_train(emb, idx_flat,
                                  jnp.zeros((b * tseq,), jnp.int32))
        return logits.reshape(b, tseq, v), None
    tgt_flat = targets.reshape(b * tseq).astype(jnp.int32)
    logits, loss = _bigram_train(emb, idx_flat, tgt_flat)
    return logits, loss


# speedup vs baseline: 1.3766x; 1.3766x over previous
import jax
import jax.numpy as jnp
from jax import lax
from jax.experimental import pallas as pl
from jax.experimental.pallas import tpu as pltpu


def _round_up(x, m):
    return (x + m - 1) // m * m


_TILE = 512
_UNROLL = 8
_LSE_TILE = 512


def _lse_tab_kernel(emb_ref, lse_ref):
    x = emb_ref[...]
    mx = jnp.max(x, axis=1, keepdims=True)
    lse_ref[...] = mx + jnp.log(jnp.sum(jnp.exp(x - mx), axis=1, keepdims=True))


def _gather_kernel(idx_sref, tgt_sref, nrows_sref, emb_hbm, lse_hbm,
                   logits_hbm, tileloss_ref, tab, lse3, buf,
                   tab_sem, lse_sem, out_sem):
    i = pl.program_id(0)
    j = pl.program_id(1)
    nj = pl.num_programs(1)
    tile = buf.shape[0]
    base = (i * nj + j) * tile
    nrows = nrows_sref[0]

    @pl.when(j == 0)
    def _load_table():
        cp1 = pltpu.make_async_copy(emb_hbm, tab.at[:, 0, :], tab_sem)
        cp2 = pltpu.make_async_copy(lse_hbm, lse3.at[:, :, 0], lse_sem)
        cp1.start()
        cp2.start()
        cp1.wait()
        cp2.wait()

    lane = lax.broadcasted_iota(jnp.int32, (1, 128), 1)

    def _chunk(o, acc):
        for u in range(_UNROLL):
            m = o * _UNROLL + u
            s = idx_sref[base + m]
            t = tgt_sref[base + m]
            buf[m] = tab[s]
            c0 = pl.multiple_of((t >> 7) << 7, 128)
            chunk = tab[s, :, pl.ds(c0, 128)]
            tl = jnp.sum(jnp.where(lane == (t & 127), chunk, 0.0),
                         axis=1, keepdims=True)
            w = jnp.where(base + m < nrows, 1.0, 0.0).astype(jnp.float32)
            acc = acc + (lse3[s] - tl) * w
        return acc

    acc = lax.fori_loop(0, tile // _UNROLL, _chunk,
                        jnp.zeros((1, 1), jnp.float32))

    out_cp = pltpu.make_async_copy(
        buf.at[:, 0, :], logits_hbm.at[pl.ds(base, tile), :], out_sem)
    out_cp.start()
    tileloss_ref[...] = acc[None]
    out_cp.wait()


@jax.jit
def _bigram_train(emb, idx_flat, tgt_flat):
    bt = idx_flat.shape[0]
    v = emb.shape[1]
    bt_pad = _round_up(bt, 2 * _TILE)
    nt = bt_pad // _TILE
    nj = nt // 2

    idx_pad = jnp.zeros((bt_pad,), jnp.int32).at[:bt].set(idx_flat)
    tgt_pad = jnp.zeros((bt_pad,), jnp.int32).at[:bt].set(tgt_flat)

    lse_tile = min(_LSE_TILE, max(v // 2, 8))
    nlj = max(v // lse_tile // 2, 1)
    lse_tab = pl.pallas_call(
        _lse_tab_kernel,
        grid=(2, nlj),
        in_specs=[pl.BlockSpec((lse_tile, v), lambda i, j: (i * nlj + j, 0))],
        out_specs=pl.BlockSpec((lse_tile, 1), lambda i, j: (i * nlj + j, 0)),
        out_shape=jax.ShapeDtypeStruct((v, 1), jnp.float32),
        compiler_params=pltpu.CompilerParams(
            dimension_semantics=("parallel", "arbitrary"),
        ),
    )(emb)

    grid_spec = pltpu.PrefetchScalarGridSpec(
        num_scalar_prefetch=3,
        grid=(2, nj),
        in_specs=[
            pl.BlockSpec(memory_space=pl.ANY),
            pl.BlockSpec(memory_space=pl.ANY),
        ],
        out_specs=[
            pl.BlockSpec(memory_space=pl.ANY),
            pl.BlockSpec((1, 1, 1), lambda i, j, *_: (i * nj + j, 0, 0)),
        ],
        scratch_shapes=[
            pltpu.VMEM((v, 1, v), jnp.float32),
            pltpu.VMEM((v, 1, 1), jnp.float32),
            pltpu.VMEM((_TILE, 1, v), jnp.float32),
            pltpu.SemaphoreType.DMA,
            pltpu.SemaphoreType.DMA,
            pltpu.SemaphoreType.DMA,
        ],
    )

    nrows = jnp.full((1,), bt, jnp.int32)
    logits, tileloss = pl.pallas_call(
        _gather_kernel,
        grid_spec=grid_spec,
        out_shape=(
            jax.ShapeDtypeStruct((bt_pad, v), emb.dtype),
            jax.ShapeDtypeStruct((nt, 1, 1), jnp.float32),
        ),
        compiler_params=pltpu.CompilerParams(
            dimension_semantics=("parallel", "arbitrary"),
            vmem_limit_bytes=44 * 1024 * 1024,
        ),
    )(idx_pad, tgt_pad, nrows, emb, lse_tab)

    loss = jnp.sum(tileloss) / bt
    return logits[:bt], loss


def kernel(emb, idx, targets):
    b, tseq = idx.shape
    v = emb.shape[1]
    idx_flat = idx.reshape(b * tseq).astype(jnp.int32)
    if targets is None:
        logits, _ = _bigram_train(emb, idx_flat,
                                  jnp.zeros((b * tseq,), jnp.int32))
        return logits.reshape(b, tseq, v), None
    tgt_flat = targets.reshape(b * tseq).astype(jnp.int32)
    logits, loss = _bigram_train(emb, idx_flat, tgt_flat)
    return logits, loss


# speedup vs baseline: 1.5112x; 1.0978x over previous
import jax
import jax.numpy as jnp
from jax import lax
from jax.experimental import pallas as pl
from jax.experimental.pallas import tpu as pltpu


def _round_up(x, m):
    return (x + m - 1) // m * m


_TILE = 512
_LSE_TILE = 512


def _lse_tab_kernel(emb_ref, lse_ref):
    x = emb_ref[...]
    mx = jnp.max(x, axis=1, keepdims=True)
    lse_ref[...] = mx + jnp.log(jnp.sum(jnp.exp(x - mx), axis=1, keepdims=True))


def _gather_kernel(idx_sref, tgt_sref, nrows_sref, emb_ref, lse_ref,
                   logits_ref, tileloss_ref, tab, lse3):
    i = pl.program_id(0)
    j = pl.program_id(1)
    nj = pl.num_programs(1)
    tile, v = logits_ref.shape
    base = (i * nj + j) * tile
    nrows = nrows_sref[0]

    @pl.when(j == 0)
    def _build_tab():
        def _relay(c, carry):
            tab[pl.ds(c * 128, 128)] = emb_ref[pl.ds(c * 128, 128), :][:, None, :]
            return carry
        lax.fori_loop(0, v // 128, _relay, 0)
        lse3[...] = lse_ref[...][:, :, None]

    lane = lax.broadcasted_iota(jnp.int32, (1, 128), 1)

    def _group(g, acc):
        rows = []
        for u in range(8):
            m = g * 8 + u
            s = idx_sref[base + m]
            t = tgt_sref[base + m]
            rows.append(tab[s])
            c0 = pl.multiple_of((t >> 7) << 7, 128)
            chunk = tab[s, :, pl.ds(c0, 128)]
            tl = jnp.sum(jnp.where(lane == (t & 127), chunk, 0.0),
                         axis=1, keepdims=True)
            w = jnp.where(base + m < nrows, 1.0, 0.0).astype(jnp.float32)
            acc = acc + (lse3[s] - tl) * w
        logits_ref[pl.ds(pl.multiple_of(g * 8, 8), 8), :] = jnp.concatenate(
            rows, axis=0)
        return acc

    acc = lax.fori_loop(0, tile // 8, _group, jnp.zeros((1, 1), jnp.float32))
    tileloss_ref[...] = acc[None]


@jax.jit
def _bigram_train(emb, idx_flat, tgt_flat):
    bt = idx_flat.shape[0]
    v = emb.shape[1]
    bt_pad = _round_up(bt, 2 * _TILE)
    nt = bt_pad // _TILE
    nj = nt // 2

    idx_pad = jnp.zeros((bt_pad,), jnp.int32).at[:bt].set(idx_flat)
    tgt_pad = jnp.zeros((bt_pad,), jnp.int32).at[:bt].set(tgt_flat)

    lse_tile = min(_LSE_TILE, max(v // 2, 8))
    nlj = max(v // lse_tile // 2, 1)
    lse_tab = pl.pallas_call(
        _lse_tab_kernel,
        grid=(2, nlj),
        in_specs=[pl.BlockSpec((lse_tile, v), lambda i, j: (i * nlj + j, 0))],
        out_specs=pl.BlockSpec((lse_tile, 1), lambda i, j: (i * nlj + j, 0)),
        out_shape=jax.ShapeDtypeStruct((v, 1), jnp.float32),
        compiler_params=pltpu.CompilerParams(
            dimension_semantics=("parallel", "arbitrary"),
        ),
    )(emb)

    grid_spec = pltpu.PrefetchScalarGridSpec(
        num_scalar_prefetch=3,
        grid=(2, nj),
        in_specs=[
            pl.BlockSpec((v, v), lambda i, j, *_: (0, 0)),
            pl.BlockSpec((v, 1), lambda i, j, *_: (0, 0)),
        ],
        out_specs=[
            pl.BlockSpec((_TILE, v), lambda i, j, *_: (i * nj + j, 0)),
            pl.BlockSpec((1, 1, 1), lambda i, j, *_: (i * nj + j, 0, 0)),
        ],
        scratch_shapes=[
            pltpu.VMEM((v, 1, v), jnp.float32),
            pltpu.VMEM((v, 1, 1), jnp.float32),
        ],
    )

    nrows = jnp.full((1,), bt, jnp.int32)
    logits, tileloss = pl.pallas_call(
        _gather_kernel,
        grid_spec=grid_spec,
        out_shape=(
            jax.ShapeDtypeStruct((bt_pad, v), emb.dtype),
            jax.ShapeDtypeStruct((nt, 1, 1), jnp.float32),
        ),
        compiler_params=pltpu.CompilerParams(
            dimension_semantics=("parallel", "arbitrary"),
            vmem_limit_bytes=50 * 1024 * 1024,
        ),
    )(idx_pad, tgt_pad, nrows, emb, lse_tab)

    loss = jnp.sum(tileloss) / bt
    return logits[:bt], loss


def kernel(emb, idx, targets):
    b, tseq = idx.shape
    v = emb.shape[1]
    idx_flat = idx.reshape(b * tseq).astype(jnp.int32)
    if targets is None:
        logits, _ = _bigram_train(emb, idx_flat,
                                  jnp.zeros((b * tseq,), jnp.int32))
        return logits.reshape(b, tseq, v), None
    tgt_flat = targets.reshape(b * tseq).astype(jnp.int32)
    logits, loss = _bigram_train(emb, idx_flat, tgt_flat)
    return logits, loss


# speedup vs baseline: 2.8435x; 1.8817x over previous
import jax
import jax.numpy as jnp
from jax import lax
from jax.experimental import pallas as pl
from jax.experimental.pallas import tpu as pltpu


def _round_up(x, m):
    return (x + m - 1) // m * m


_TILE = 512
_ROT = 8


def _gather_kernel(idx_sref, tgt_sref, nrows_sref, emb_ref,
                   logits_ref, tileloss_ref, tab, lse3):
    i = pl.program_id(0)
    j = pl.program_id(1)
    nj = pl.num_programs(1)
    tile, v = logits_ref.shape
    base = (i * nj + j) * tile
    nrows = nrows_sref[0]

    @pl.when(j == 0)
    def _build_tab():
        def _relay(c, carry):
            tab[pl.ds(c * 128, 128)] = emb_ref[pl.ds(c * 128, 128), :][:, None, :]
            return carry
        lax.fori_loop(0, v // 128, _relay, 0)

        def _lse_chunk(c, carry):
            x = emb_ref[pl.ds(c * 256, 256), :]
            mx = jnp.max(x, axis=1, keepdims=True)
            lse = mx + jnp.log(jnp.sum(jnp.exp(x - mx), axis=1, keepdims=True))
            lse3[pl.ds(c * 256, 256)] = lse[:, :, None]
            return carry
        lax.fori_loop(0, v // 256, _lse_chunk, 0)

    lane = lax.broadcasted_iota(jnp.int32, (1, 128), 1)

    def _group(g, carry):
        accs = list(carry)
        rows = []
        for u in range(_ROT):
            m = g * _ROT + u
            s = idx_sref[base + m]
            t = tgt_sref[base + m]
            rows.append(tab[s])
            c0 = pl.multiple_of((t >> 7) << 7, 128)
            chunk = tab[s, :, pl.ds(c0, 128)]
            valid = base + m < nrows
            msk = jnp.logical_and(lane == (t & 127), valid)
            accs[u] = accs[u] + jnp.where(msk, chunk, 0.0)
            accs[_ROT + u] = accs[_ROT + u] + jnp.where(
                valid, lse3[s], jnp.zeros((1, 1), jnp.float32))
        logits_ref[pl.ds(pl.multiple_of(g * _ROT, 8), 8), :] = jnp.concatenate(
            rows, axis=0)
        return tuple(accs)

    init = tuple(jnp.zeros((1, 128), jnp.float32) for _ in range(_ROT)) + \
           tuple(jnp.zeros((1, 1), jnp.float32) for _ in range(_ROT))
    accs = lax.fori_loop(0, tile // _ROT, _group, init)

    tl_total = jnp.sum(sum(accs[:_ROT]), axis=1, keepdims=True)
    lse_total = sum(accs[_ROT:])
    tileloss_ref[...] = (lse_total - tl_total)[None]


@jax.jit
def _bigram_train(emb, idx_flat, tgt_flat):
    bt = idx_flat.shape[0]
    v = emb.shape[1]
    bt_pad = _round_up(bt, 2 * _TILE)
    nt = bt_pad // _TILE
    nj = nt // 2

    idx_pad = jnp.zeros((bt_pad,), jnp.int32).at[:bt].set(idx_flat)
    tgt_pad = jnp.zeros((bt_pad,), jnp.int32).at[:bt].set(tgt_flat)

    grid_spec = pltpu.PrefetchScalarGridSpec(
        num_scalar_prefetch=3,
        grid=(2, nj),
        in_specs=[
            pl.BlockSpec((v, v), lambda i, j, *_: (0, 0)),
        ],
        out_specs=[
            pl.BlockSpec((_TILE, v), lambda i, j, *_: (i * nj + j, 0)),
            pl.BlockSpec((1, 1, 1), lambda i, j, *_: (i * nj + j, 0, 0)),
        ],
        scratch_shapes=[
            pltpu.VMEM((v, 1, v), jnp.float32),
            pltpu.VMEM((v, 1, 1), jnp.float32),
        ],
    )

    nrows = jnp.full((1,), bt, jnp.int32)
    logits, tileloss = pl.pallas_call(
        _gather_kernel,
        grid_spec=grid_spec,
        out_shape=(
            jax.ShapeDtypeStruct((bt_pad, v), emb.dtype),
            jax.ShapeDtypeStruct((nt, 1, 1), jnp.float32),
        ),
        compiler_params=pltpu.CompilerParams(
            dimension_semantics=("parallel", "arbitrary"),
            vmem_limit_bytes=50 * 1024 * 1024,
        ),
    )(idx_pad, tgt_pad, nrows, emb)

    loss = jnp.sum(tileloss) / bt
    return logits[:bt], loss


def kernel(emb, idx, targets):
    b, tseq = idx.shape
    v = emb.shape[1]
    idx_flat = idx.reshape(b * tseq).astype(jnp.int32)
    if targets is None:
        logits, _ = _bigram_train(emb, idx_flat,
                                  jnp.zeros((b * tseq,), jnp.int32))
        return logits.reshape(b, tseq, v), None
    tgt_flat = targets.reshape(b * tseq).astype(jnp.int32)
    logits, loss = _bigram_train(emb, idx_flat, tgt_flat)
    return logits, loss


# speedup vs baseline: 2.9898x; 1.0514x over previous
import jax
import jax.numpy as jnp
from jax import lax
from jax.experimental import pallas as pl
from jax.experimental.pallas import tpu as pltpu


def _round_up(x, m):
    return (x + m - 1) // m * m


_TILE = 512
_ROT = 8


def _gather_kernel(idx_sref, tgt_sref, nrows_sref, emb_ref,
                   logits_ref, tileloss_ref, tab, lse3, *, padded):
    i = pl.program_id(0)
    j = pl.program_id(1)
    nj = pl.num_programs(1)
    tile, v = logits_ref.shape
    base = (i * nj + j) * tile
    nrows = nrows_sref[0]

    @pl.when(j == 0)
    def _build_tab():
        def _relay(c, carry):
            tab[pl.ds(c * 128, 128)] = emb_ref[pl.ds(c * 128, 128), :][:, None, :]
            return carry
        lax.fori_loop(0, v // 128, _relay, 0)

        def _lse_chunk(c, carry):
            x = emb_ref[pl.ds(c * 256, 256), :]
            mx = jnp.max(x, axis=1, keepdims=True)
            lse = mx + jnp.log(jnp.sum(jnp.exp(x - mx), axis=1, keepdims=True))
            lse3[pl.ds(c * 256, 256)] = lse[:, :, None]
            return carry
        lax.fori_loop(0, v // 256, _lse_chunk, 0)

    lane = lax.broadcasted_iota(jnp.int32, (1, 128), 1)

    def _group(g, carry):
        accs = list(carry)
        rows = []
        for u in range(_ROT):
            m = g * _ROT + u
            s = idx_sref[base + m]
            t = tgt_sref[base + m]
            rows.append(tab[s])
            c0 = pl.multiple_of((t >> 7) << 7, 128)
            chunk = tab[s, :, pl.ds(c0, 128)]
            if padded:
                valid = base + m < nrows
                msk = jnp.logical_and(lane == (t & 127), valid)
                accs[u] = accs[u] + jnp.where(msk, chunk, 0.0)
                accs[_ROT + u] = accs[_ROT + u] + jnp.where(
                    valid, lse3[s], jnp.zeros((1, 1), jnp.float32))
            else:
                accs[u] = accs[u] + jnp.where(lane == (t & 127), chunk, 0.0)
                accs[_ROT + u] = accs[_ROT + u] + lse3[s]
        logits_ref[pl.ds(pl.multiple_of(g * _ROT, 8), 8), :] = jnp.concatenate(
            rows, axis=0)
        return tuple(accs)

    init = tuple(jnp.zeros((1, 128), jnp.float32) for _ in range(_ROT)) + \
           tuple(jnp.zeros((1, 1), jnp.float32) for _ in range(_ROT))
    accs = lax.fori_loop(0, tile // _ROT, _group, init)

    tl_total = jnp.sum(sum(accs[:_ROT]), axis=1, keepdims=True)
    lse_total = sum(accs[_ROT:])
    tileloss_ref[...] = (lse_total - tl_total)[None]


@jax.jit
def _bigram_train(emb, idx_flat, tgt_flat):
    bt = idx_flat.shape[0]
    v = emb.shape[1]
    bt_pad = _round_up(bt, 2 * _TILE)
    nt = bt_pad // _TILE
    nj = nt // 2

    idx_pad = jnp.zeros((bt_pad,), jnp.int32).at[:bt].set(idx_flat)
    tgt_pad = jnp.zeros((bt_pad,), jnp.int32).at[:bt].set(tgt_flat)

    grid_spec = pltpu.PrefetchScalarGridSpec(
        num_scalar_prefetch=3,
        grid=(2, nj),
        in_specs=[
            pl.BlockSpec((v, v), lambda i, j, *_: (0, 0)),
        ],
        out_specs=[
            pl.BlockSpec((_TILE, v), lambda i, j, *_: (i * nj + j, 0)),
            pl.BlockSpec((1, 1, 1), lambda i, j, *_: (i * nj + j, 0, 0)),
        ],
        scratch_shapes=[
            pltpu.VMEM((v, 1, v), jnp.float32),
            pltpu.VMEM((v, 1, 1), jnp.float32),
        ],
    )

    nrows = jnp.full((1,), bt, jnp.int32)
    import functools as _ft
    logits, tileloss = pl.pallas_call(
        _ft.partial(_gather_kernel, padded=(bt != bt_pad)),
        grid_spec=grid_spec,
        out_shape=(
            jax.ShapeDtypeStruct((bt_pad, v), emb.dtype),
            jax.ShapeDtypeStruct((nt, 1, 1), jnp.float32),
        ),
        compiler_params=pltpu.CompilerParams(
            dimension_semantics=("parallel", "arbitrary"),
            vmem_limit_bytes=50 * 1024 * 1024,
        ),
    )(idx_pad, tgt_pad, nrows, emb)

    loss = jnp.sum(tileloss) / bt
    return logits[:bt], loss


def kernel(emb, idx, targets):
    b, tseq = idx.shape
    v = emb.shape[1]
    idx_flat = idx.reshape(b * tseq).astype(jnp.int32)
    if targets is None:
        logits, _ = _bigram_train(emb, idx_flat,
                                  jnp.zeros((b * tseq,), jnp.int32))
        return logits.reshape(b, tseq, v), None
    tgt_flat = targets.reshape(b * tseq).astype(jnp.int32)
    logits, loss = _bigram_train(emb, idx_flat, tgt_flat)
    return logits, loss


# speedup vs baseline: 3.0562x; 1.0222x over previous
import jax
import jax.numpy as jnp
from jax import lax
from jax.experimental import pallas as pl
from jax.experimental.pallas import tpu as pltpu


def _round_up(x, m):
    return (x + m - 1) // m * m


_TILE = 512
_ROT = 16


def _gather_kernel(idx_sref, tgt_sref, nrows_sref, emb_ref,
                   logits_ref, tileloss_ref, tab, lse3, *, padded):
    i = pl.program_id(0)
    j = pl.program_id(1)
    nj = pl.num_programs(1)
    tile, v = logits_ref.shape
    base = (i * nj + j) * tile
    nrows = nrows_sref[0]

    @pl.when(j == 0)
    def _build_tab():
        def _relay(c, carry):
            tab[pl.ds(c * 128, 128)] = emb_ref[pl.ds(c * 128, 128), :][:, None, :]
            return carry
        lax.fori_loop(0, v // 128, _relay, 0)

        def _lse_chunk(c, carry):
            x = emb_ref[pl.ds(c * 256, 256), :]
            mx = jnp.max(x, axis=1, keepdims=True)
            lse = mx + jnp.log(jnp.sum(jnp.exp(x - mx), axis=1, keepdims=True))
            lse3[pl.ds(c * 256, 256)] = lse[:, :, None]
            return carry
        lax.fori_loop(0, v // 256, _lse_chunk, 0)

    lane = lax.broadcasted_iota(jnp.int32, (1, 128), 1)

    def _group(g, carry):
        accs = list(carry)
        rows = []
        for u in range(_ROT):
            m = g * _ROT + u
            s = idx_sref[base + m]
            t = tgt_sref[base + m]
            rows.append(tab[s])
            c0 = pl.multiple_of((t >> 7) << 7, 128)
            chunk = tab[s, :, pl.ds(c0, 128)]
            if padded:
                valid = base + m < nrows
                msk = jnp.logical_and(lane == (t & 127), valid)
                accs[u] = accs[u] + jnp.where(msk, chunk, 0.0)
                accs[_ROT + u] = accs[_ROT + u] + jnp.where(
                    valid, lse3[s], jnp.zeros((1, 1), jnp.float32))
            else:
                accs[u] = accs[u] + jnp.where(lane == (t & 127), chunk, 0.0)
                accs[_ROT + u] = accs[_ROT + u] + lse3[s]
        logits_ref[pl.ds(pl.multiple_of(g * _ROT, 8), 8), :] = jnp.concatenate(
            rows[:8], axis=0)
        logits_ref[pl.ds(pl.multiple_of(g * _ROT + 8, 8), 8), :] = jnp.concatenate(
            rows[8:], axis=0)
        return tuple(accs)

    init = tuple(jnp.zeros((1, 128), jnp.float32) for _ in range(_ROT)) + \
           tuple(jnp.zeros((1, 1), jnp.float32) for _ in range(_ROT))
    accs = lax.fori_loop(0, tile // _ROT, _group, init)

    tl_total = jnp.sum(sum(accs[:_ROT]), axis=1, keepdims=True)
    lse_total = sum(accs[_ROT:])
    tileloss_ref[...] = (lse_total - tl_total)[None]


@jax.jit
def _bigram_train(emb, idx_flat, tgt_flat):
    bt = idx_flat.shape[0]
    v = emb.shape[1]
    bt_pad = _round_up(bt, 2 * _TILE)
    nt = bt_pad // _TILE
    nj = nt // 2

    idx_pad = jnp.zeros((bt_pad,), jnp.int32).at[:bt].set(idx_flat)
    tgt_pad = jnp.zeros((bt_pad,), jnp.int32).at[:bt].set(tgt_flat)

    grid_spec = pltpu.PrefetchScalarGridSpec(
        num_scalar_prefetch=3,
        grid=(2, nj),
        in_specs=[
            pl.BlockSpec((v, v), lambda i, j, *_: (0, 0)),
        ],
        out_specs=[
            pl.BlockSpec((_TILE, v), lambda i, j, *_: (i * nj + j, 0)),
            pl.BlockSpec((1, 1, 1), lambda i, j, *_: (i * nj + j, 0, 0)),
        ],
        scratch_shapes=[
            pltpu.VMEM((v, 1, v), jnp.float32),
            pltpu.VMEM((v, 1, 1), jnp.float32),
        ],
    )

    nrows = jnp.full((1,), bt, jnp.int32)
    import functools as _ft
    logits, tileloss = pl.pallas_call(
        _ft.partial(_gather_kernel, padded=(bt != bt_pad)),
        grid_spec=grid_spec,
        out_shape=(
            jax.ShapeDtypeStruct((bt_pad, v), emb.dtype),
            jax.ShapeDtypeStruct((nt, 1, 1), jnp.float32),
        ),
        compiler_params=pltpu.CompilerParams(
            dimension_semantics=("parallel", "arbitrary"),
            vmem_limit_bytes=50 * 1024 * 1024,
        ),
    )(idx_pad, tgt_pad, nrows, emb)

    loss = jnp.sum(tileloss) / bt
    return logits[:bt], loss


def kernel(emb, idx, targets):
    b, tseq = idx.shape
    v = emb.shape[1]
    idx_flat = idx.reshape(b * tseq).astype(jnp.int32)
    if targets is None:
        logits, _ = _bigram_train(emb, idx_flat,
                                  jnp.zeros((b * tseq,), jnp.int32))
        return logits.reshape(b, tseq, v), None
    tgt_flat = targets.reshape(b * tseq).astype(jnp.int32)
    logits, loss = _bigram_train(emb, idx_flat, tgt_flat)
    return logits, loss


# speedup vs baseline: 3.2281x; 1.0562x over previous
import jax
import jax.numpy as jnp
from jax import lax
from jax.experimental import pallas as pl
from jax.experimental.pallas import tpu as pltpu


def _round_up(x, m):
    return (x + m - 1) // m * m


_TILE = 512
_ROT = 16


def _gather_kernel(idx_sref, tgt_sref, nrows_sref, emb_ref,
                   logits_ref, tileloss_ref, tab, lse3, *, padded):
    i = pl.program_id(0)
    j = pl.program_id(1)
    nj = pl.num_programs(1)
    tile, v = logits_ref.shape
    base = (i * nj + j) * tile
    nrows = nrows_sref[0]

    @pl.when(j == 0)
    def _build_tab():
        def _relay(c, carry):
            tab[pl.ds(c * 128, 128)] = emb_ref[pl.ds(c * 128, 128), :][:, None, :]
            return carry
        lax.fori_loop(0, v // 128, _relay, 0)

        def _lse_chunk(c, carry):
            x = emb_ref[pl.ds(c * 256, 256), :]
            mx = jnp.max(x, axis=1, keepdims=True)
            lse = mx + jnp.log(jnp.sum(jnp.exp(x - mx), axis=1, keepdims=True))
            lse3[pl.ds(c * 256, 256)] = lse[:, :, None]
            return carry
        lax.fori_loop(0, v // 256, _lse_chunk, 0)

    lane = lax.broadcasted_iota(jnp.int32, (1, 128), 1)

    def _group(g, carry):
        accs = list(carry)
        rows = []
        for u in range(_ROT):
            m = g * _ROT + u
            s = idx_sref[base + m]
            t = tgt_sref[base + m]
            rows.append(tab[s])
            c0 = pl.multiple_of((t >> 7) << 7, 128)
            chunk = tab[s, :, pl.ds(c0, 128)]
            if padded:
                valid = base + m < nrows
                msk = jnp.logical_and(lane == (t & 127), valid)
                accs[u] = accs[u] + jnp.where(msk, chunk, 0.0)
                accs[_ROT + u] = accs[_ROT + u] + jnp.where(
                    valid, lse3[s], jnp.zeros((1, 1), jnp.float32))
            else:
                accs[u] = accs[u] + jnp.where(lane == (t & 127), chunk, 0.0)
                accs[_ROT + u] = accs[_ROT + u] + lse3[s]
        logits_ref[pl.ds(pl.multiple_of(g * _ROT, 8), 8), :] = jnp.concatenate(
            rows[:8], axis=0)
        logits_ref[pl.ds(pl.multiple_of(g * _ROT + 8, 8), 8), :] = jnp.concatenate(
            rows[8:], axis=0)
        return tuple(accs)

    init = tuple(jnp.zeros((1, 128), jnp.float32) for _ in range(_ROT)) + \
           tuple(jnp.zeros((1, 1), jnp.float32) for _ in range(_ROT))
    accs = lax.fori_loop(0, tile // _ROT, _group, init, unroll=True)

    tl_total = jnp.sum(sum(accs[:_ROT]), axis=1, keepdims=True)
    lse_total = sum(accs[_ROT:])
    tileloss_ref[...] = (lse_total - tl_total)[None]


@jax.jit
def _bigram_train(emb, idx_flat, tgt_flat):
    bt = idx_flat.shape[0]
    v = emb.shape[1]
    bt_pad = _round_up(bt, 2 * _TILE)
    nt = bt_pad // _TILE
    nj = nt // 2

    idx_pad = jnp.zeros((bt_pad,), jnp.int32).at[:bt].set(idx_flat)
    tgt_pad = jnp.zeros((bt_pad,), jnp.int32).at[:bt].set(tgt_flat)

    grid_spec = pltpu.PrefetchScalarGridSpec(
        num_scalar_prefetch=3,
        grid=(2, nj),
        in_specs=[
            pl.BlockSpec((v, v), lambda i, j, *_: (0, 0)),
        ],
        out_specs=[
            pl.BlockSpec((_TILE, v), lambda i, j, *_: (i * nj + j, 0)),
            pl.BlockSpec((1, 1, 1), lambda i, j, *_: (i * nj + j, 0, 0)),
        ],
        scratch_shapes=[
            pltpu.VMEM((v, 1, v), jnp.float32),
            pltpu.VMEM((v, 1, 1), jnp.float32),
        ],
    )

    nrows = jnp.full((1,), bt, jnp.int32)
    import functools as _ft
    logits, tileloss = pl.pallas_call(
        _ft.partial(_gather_kernel, padded=(bt != bt_pad)),
        grid_spec=grid_spec,
        out_shape=(
            jax.ShapeDtypeStruct((bt_pad, v), emb.dtype),
            jax.ShapeDtypeStruct((nt, 1, 1), jnp.float32),
        ),
        compiler_params=pltpu.CompilerParams(
            dimension_semantics=("parallel", "arbitrary"),
            vmem_limit_bytes=50 * 1024 * 1024,
        ),
    )(idx_pad, tgt_pad, nrows, emb)

    loss = jnp.sum(tileloss) / bt
    return logits[:bt], loss


def kernel(emb, idx, targets):
    b, tseq = idx.shape
    v = emb.shape[1]
    idx_flat = idx.reshape(b * tseq).astype(jnp.int32)
    if targets is None:
        logits, _ = _bigram_train(emb, idx_flat,
                                  jnp.zeros((b * tseq,), jnp.int32))
        return logits.reshape(b, tseq, v), None
    tgt_flat = targets.reshape(b * tseq).astype(jnp.int32)
    logits, loss = _bigram_train(emb, idx_flat, tgt_flat)
    return logits, loss
